# baseline (speedup 1.0000x reference)
import functools

import jax
import jax.numpy as jnp
from jax import lax
from jax.experimental import pallas as pl
from jax.experimental.pallas import tpu as pltpu
from jax.experimental.pallas import tpu_sc as plsc

NUM_CORES = 2
NUM_SUBCORES = 16
NW = NUM_CORES * NUM_SUBCORES

L = 128
T = 50
DIM = 64


@jax.jit
def _emb_gather(idx, table):
    b_per_w = T * L
    mesh = plsc.VectorSubcoreMesh(core_axis_name="c", subcore_axis_name="s")

    @functools.partial(
        pl.kernel,
        mesh=mesh,
        out_type=jax.ShapeDtypeStruct((NW * b_per_w, DIM), jnp.float32),
        scratch_types=[
            pltpu.VMEM((T, L), jnp.int32),
            pltpu.VMEM((2, L, DIM), jnp.float32),
        ] + [pltpu.SemaphoreType.DMA] * 4,
        compiler_params=pltpu.CompilerParams(use_tc_tiling_on_sc=False),
    )
    def emb_kernel(idx_hbm, table_hbm, out_hbm, idx_v, gbuf,
                   gsem0, gsem1, wsem0, wsem1):
        wid = lax.axis_index("s") * NUM_CORES + lax.axis_index("c")
        gsems = (gsem0, gsem1)
        wsems = (wsem0, wsem1)
        base = wid * b_per_w
        pltpu.sync_copy(idx_hbm.at[wid], idx_v)
        pltpu.async_copy(table_hbm.at[idx_v.at[0]], gbuf.at[0], gsem0)

        def out_dst(t):
            return out_hbm.at[pl.ds(base + t * L, L)]

        def round_body(rr, carry):
            for b in range(2):
                t = rr * 2 + b
                pltpu.make_async_copy(
                    table_hbm.at[idx_v.at[t]], gbuf.at[b], gsems[b]
                ).wait()

                @pl.when(t < T - 1)
                def _():
                    pltpu.async_copy(
                        table_hbm.at[idx_v.at[t + 1]], gbuf.at[1 - b],
                        gsems[1 - b],
                    )

                @pl.when(t >= 2)
                def _():
                    pltpu.make_async_copy(
                        gbuf.at[b], out_dst(t - 2), wsems[b]
                    ).wait()

                pltpu.make_async_copy(
                    gbuf.at[b], out_dst(t), wsems[b]
                ).start()
            return carry

        lax.fori_loop(0, T // 2, round_body, 0)

        for b in range(2):
            t = T - 2 + b
            pltpu.make_async_copy(
                gbuf.at[b], out_dst(t), wsems[b]
            ).wait()

    return emb_kernel(idx, table)


def _tc_transpose_body(in_ref, out_ref):
    blk = in_ref[...]
    even = blk[:, 0:DIM]
    odd = blk[:, DIM:2 * DIM]
    et = even.T
    ot = odd.T
    inter = jnp.stack([et, ot], axis=-1).reshape(DIM, L)
    out_ref[...] = inter.reshape(1, DIM // 8, 1, 8, L)


@jax.jit
def _tc_transpose(rows2):
    return pl.pallas_call(
        _tc_transpose_body,
        grid=(NW, T),
        in_specs=[
            pl.BlockSpec((DIM, 2 * DIM), lambda w, t: (w * T + t, 0)),
        ],
        out_specs=pl.BlockSpec(
            (1, DIM // 8, 1, 8, L), lambda w, t: (t, 0, w, 0, 0)
        ),
        out_shape=jax.ShapeDtypeStruct((T, DIM // 8, NW, 8, L), jnp.float32),
    )(rows2)


def kernel(x, weight):
    idx = x.astype(jnp.int32).T.reshape(T, NW, L).transpose(1, 0, 2)
    rows = _emb_gather(idx, weight)
    rows2 = rows.reshape(NW * T * L // 2, 2 * DIM)
    out5 = _tc_transpose(rows2)
    out = (
        out5.transpose(2, 4, 0, 1, 3)
        .reshape(NW * L, T, DIM)
    )
    return out


# speedup vs baseline: 4.6986x; 4.6986x over previous
import functools

import jax
import jax.numpy as jnp
from jax import lax
from jax.experimental import pallas as pl
from jax.experimental.pallas import tpu as pltpu
from jax.experimental.pallas import tpu_sc as plsc

NUM_CORES = 2
NUM_SUBCORES = 16
NW = NUM_CORES * NUM_SUBCORES

CHUNK = 128
NBUF = 5


@functools.partial(jax.jit, static_argnums=(2, 3))
def _emb_lookup(idx, table, nchunk, dim):
    b_total = NW * nchunk * CHUNK
    b_per_w = nchunk * CHUNK
    rounds = nchunk // NBUF

    mesh = plsc.VectorSubcoreMesh(core_axis_name="c", subcore_axis_name="s")

    @functools.partial(
        pl.kernel,
        mesh=mesh,
        out_type=jax.ShapeDtypeStruct((b_total, dim), jnp.float32),
        scratch_types=[
            pltpu.VMEM((nchunk, CHUNK), jnp.int32),
            pltpu.VMEM((NBUF, CHUNK, dim), jnp.float32),
        ] + [pltpu.SemaphoreType.DMA] * NBUF,
        compiler_params=pltpu.CompilerParams(use_tc_tiling_on_sc=False),
    )
    def emb_kernel(idx_hbm, table_hbm, out_hbm, idx_v, rows_v, *gsems):
        wid = lax.axis_index("s") * NUM_CORES + lax.axis_index("c")
        base = wid * b_per_w
        pltpu.sync_copy(idx_hbm.at[wid], idx_v)

        for b in range(NBUF):
            pltpu.async_copy(table_hbm.at[idx_v.at[b]], rows_v.at[b], gsems[b])

        def body(i, carry):
            for b in range(NBUF):
                j = i * NBUF + b
                pltpu.make_async_copy(
                    table_hbm.at[idx_v.at[j]], rows_v.at[b], gsems[b]
                ).wait()
                pltpu.sync_copy(
                    rows_v.at[b], out_hbm.at[pl.ds(base + j * CHUNK, CHUNK)]
                )

                @pl.when(i < rounds - 1)
                def _():
                    pltpu.async_copy(
                        table_hbm.at[idx_v.at[j + NBUF]], rows_v.at[b], gsems[b]
                    )

            return carry

        lax.fori_loop(0, rounds, body, 0)

    return emb_kernel(idx, table)


def kernel(x, weight):
    b, t = x.shape
    dim = weight.shape[1]
    b_total = b * t
    assert b_total % (NW * CHUNK) == 0
    nchunk = b_total // (NW * CHUNK)
    assert nchunk % NBUF == 0
    idx = x.reshape(NW, nchunk, CHUNK).astype(jnp.int32)
    rows = _emb_lookup(idx, weight, nchunk, dim)
    return rows.reshape(b, t, dim)
